# Initial kernel scaffold; baseline (speedup 1.0000x reference)
import jax
import jax.numpy as jnp
from jax.experimental import pallas as pl


def kernel(x, ffn_in, ffn_in_bias, ffn_out, ffn_out_bias, router_w, router_b):
    raise NotImplementedError("write your pallas kernel here")


# speedup vs baseline: 5.6396x; 5.6396x over previous
import functools

import jax
import jax.numpy as jnp
from jax.experimental import pallas as pl
from jax.experimental.pallas import tpu as pltpu

_E = 16
_K = 2
_LIMIT = 7.0


def _moe_kernel(x_ref, rw_ref, rb_ref, w1_ref, b1_ref, w2_ref, b2_ref,
                out_ref, aux_ref, g_ref, *, ff):
    e = pl.program_id(0)
    n = x_ref.shape[0]
    xs = x_ref[...]

    @pl.when(e == 0)
    def _router():
        logits = jax.lax.dot_general(
            xs, rw_ref[...], (((1,), (1,)), ((), ())),
            preferred_element_type=jnp.float32) + rb_ref[...]
        m = jnp.max(logits, axis=1, keepdims=True)
        ex = jnp.exp(logits - m)
        p = ex / jnp.sum(ex, axis=1, keepdims=True)

        col = jax.lax.broadcasted_iota(jnp.int32, p.shape, 1)
        big = jnp.int32(10 ** 9)
        m1 = jnp.max(p, axis=1, keepdims=True)
        i1 = jnp.min(jnp.where(p >= m1, col, big), axis=1, keepdims=True)
        h1 = col == i1
        pm = jnp.where(h1, -jnp.inf, p)
        m2 = jnp.max(pm, axis=1, keepdims=True)
        i2 = jnp.min(jnp.where(pm >= m2, col, big), axis=1, keepdims=True)
        h2 = col == i2

        denom = m1 + m2 + 1e-9
        g_ref[...] = jnp.where(h1 | h2, p, 0.0) / denom

        importance = jnp.mean(p, axis=0)
        load = jnp.mean(h1.astype(jnp.float32), axis=0)
        aux_ref[...] = (_E * jnp.sum(importance * load)).reshape(1, 1)
        out_ref[...] = jnp.zeros_like(out_ref)

    w1 = w1_ref[0]
    u = jnp.dot(xs, w1[:, :ff], preferred_element_type=jnp.float32)
    v = jnp.dot(xs, w1[:, ff:], preferred_element_type=jnp.float32)
    b1 = b1_ref[0]
    up = jnp.clip(u + b1[:, :ff], -_LIMIT, _LIMIT)
    gate = jnp.clip(v + b1[:, ff:], -_LIMIT, _LIMIT)
    act = gate * jax.nn.sigmoid(gate) * up

    z = jnp.dot(act, w2_ref[0], preferred_element_type=jnp.float32) + b2_ref[0]

    col = jax.lax.broadcasted_iota(jnp.int32, g_ref.shape, 1)
    g = jnp.sum(jnp.where(col == e, g_ref[...], 0.0), axis=1, keepdims=True)
    out_ref[...] += g * z


@jax.jit
def kernel(x, ffn_in, ffn_in_bias, ffn_out, ffn_out_bias, router_w, router_b):
    b, t, h = x.shape
    e, _, ff2 = ffn_in.shape
    ff = ff2 // 2
    n = b * t
    xf = x.reshape(n, h)

    out, aux = pl.pallas_call(
        functools.partial(_moe_kernel, ff=ff),
        grid=(e,),
        in_specs=[
            pl.BlockSpec((n, h), lambda i: (0, 0)),
            pl.BlockSpec((e, h), lambda i: (0, 0)),
            pl.BlockSpec((1, e), lambda i: (0, 0)),
            pl.BlockSpec((1, h, ff2), lambda i: (i, 0, 0)),
            pl.BlockSpec((1, 1, ff2), lambda i: (i, 0, 0)),
            pl.BlockSpec((1, ff, h), lambda i: (i, 0, 0)),
            pl.BlockSpec((1, 1, h), lambda i: (i, 0, 0)),
        ],
        out_specs=[
            pl.BlockSpec((n, h), lambda i: (0, 0)),
            pl.BlockSpec((1, 1), lambda i: (0, 0)),
        ],
        out_shape=[
            jax.ShapeDtypeStruct((n, h), jnp.float32),
            jax.ShapeDtypeStruct((1, 1), jnp.float32),
        ],
        scratch_shapes=[pltpu.VMEM((n, e), jnp.float32)],
    )(xf, router_w, router_b.reshape(1, e), ffn_in,
      ffn_in_bias.reshape(e, 1, ff2), ffn_out, ffn_out_bias.reshape(e, 1, h))

    return out.reshape(b, t, h), aux[0, 0]
